# baseline (speedup 1.0000x reference)
import functools

import jax
import jax.numpy as jnp
from jax import lax
from jax.experimental import pallas as pl
from jax.experimental.pallas import tpu as pltpu
from jax.experimental.pallas import tpu_sc as plsc

_NC = 2
_NS = 16
_NW = _NC * _NS
_CHUNK = 512
_NBUF = 2


def _make_emb(N, V, D):
    n_per_w = N // _NW
    n_chunks = n_per_w // _CHUNK
    assert n_per_w * _NW == N and n_chunks * _CHUNK == n_per_w
    assert n_chunks % _NBUF == 0 and n_chunks > _NBUF
    mesh = plsc.VectorSubcoreMesh(core_axis_name="c", subcore_axis_name="s")

    @functools.partial(
        pl.kernel,
        mesh=mesh,
        compiler_params=pltpu.CompilerParams(use_tc_tiling_on_sc=False),
        out_type=jax.ShapeDtypeStruct((N, D), jnp.float32),
        scratch_types=[
            pltpu.VMEM((n_per_w,), jnp.int32),
            pltpu.VMEM((_NBUF, _CHUNK, D), jnp.float32),
            pltpu.SemaphoreType.DMA((_NBUF,)),
            pltpu.SemaphoreType.DMA((_NBUF,)),
        ],
    )
    def emb(idx_hbm, tab_hbm, out_hbm, idx_v, rows_v, gsem, ssem):
        wid = lax.axis_index("s") * _NC + lax.axis_index("c")
        base = wid * n_per_w
        pltpu.sync_copy(idx_hbm.at[pl.ds(base, n_per_w)], idx_v)

        def gather(g, b):
            return pltpu.make_async_copy(
                tab_hbm.at[pl.ds(((base + g * _CHUNK) * 977) % (1000000 - _CHUNK), _CHUNK), :],
                rows_v.at[b],
                gsem.at[b],
            )

        def store(g, b):
            return pltpu.make_async_copy(
                rows_v.at[b],
                out_hbm.at[pl.ds(base + g * _CHUNK, _CHUNK), :],
                ssem.at[b],
            )

        for b in range(_NBUF):
            gather(b, b).start()

        def outer(o, _):
            for b in range(_NBUF):
                g = o * _NBUF + b
                gather(g, b).wait()
                store(g, b).start()
                store(g, b).wait()
                gather(g + _NBUF, b).start()
            return ()

        lax.fori_loop(0, n_chunks // _NBUF - 1, outer, ())

        for b in range(_NBUF):
            g = n_chunks - _NBUF + b
            gather(g, b).wait()
            store(g, b).start()
        for b in range(_NBUF):
            store(n_chunks - _NBUF + b, b).wait()

    return emb


def kernel(input, table):
    B, H = input.shape
    V, D = table.shape
    N = B * H
    flat_idx = input.reshape(N)
    out = _make_emb(N, V, D)(flat_idx, table)
    return out.reshape(B, H, D)


# speedup vs baseline: 1.0043x; 1.0043x over previous
import functools

import jax
import jax.numpy as jnp
from jax import lax
from jax.experimental import pallas as pl
from jax.experimental.pallas import tpu as pltpu
from jax.experimental.pallas import tpu_sc as plsc

_NC = 2
_NS = 16
_NW = _NC * _NS
_CHUNK = 640
_NBUF = 2


def _make_emb(N, V, D):
    n_per_w = N // _NW
    n_chunks = n_per_w // _CHUNK
    assert n_per_w * _NW == N and n_chunks * _CHUNK == n_per_w
    assert n_chunks % _NBUF == 0 and n_chunks > _NBUF
    mesh = plsc.VectorSubcoreMesh(core_axis_name="c", subcore_axis_name="s")

    @functools.partial(
        pl.kernel,
        mesh=mesh,
        compiler_params=pltpu.CompilerParams(use_tc_tiling_on_sc=False),
        out_type=jax.ShapeDtypeStruct((N, D), jnp.float32),
        scratch_types=[
            pltpu.VMEM((n_per_w,), jnp.int32),
            pltpu.VMEM((_NBUF, _CHUNK, D), jnp.float32),
            pltpu.SemaphoreType.DMA((_NBUF,)),
            pltpu.SemaphoreType.DMA((_NBUF,)),
        ],
    )
    def emb(idx_hbm, tab_hbm, out_hbm, idx_v, rows_v, gsem, ssem):
        wid = lax.axis_index("s") * _NC + lax.axis_index("c")
        base = wid * n_per_w
        pltpu.sync_copy(idx_hbm.at[pl.ds(base, n_per_w)], idx_v)

        def gather(g, b):
            return pltpu.make_async_copy(
                tab_hbm.at[idx_v.at[pl.ds(g * _CHUNK, _CHUNK)]],
                rows_v.at[b],
                gsem.at[b],
            )

        def store(g, b):
            return pltpu.make_async_copy(
                rows_v.at[b],
                out_hbm.at[pl.ds(base + g * _CHUNK, _CHUNK), :],
                ssem.at[b],
            )

        for b in range(_NBUF):
            gather(b, b).start()

        def outer(o, _):
            for b in range(_NBUF):
                g = o * _NBUF + b
                gather(g, b).wait()
                store(g, b).start()
                store(g, b).wait()
                gather(g + _NBUF, b).start()
            return ()

        lax.fori_loop(0, n_chunks // _NBUF - 1, outer, ())

        for b in range(_NBUF):
            g = n_chunks - _NBUF + b
            gather(g, b).wait()
            store(g, b).start()
        for b in range(_NBUF):
            store(n_chunks - _NBUF + b, b).wait()

    return emb


def kernel(input, table):
    B, H = input.shape
    V, D = table.shape
    N = B * H
    flat_idx = input.reshape(N)
    out = _make_emb(N, V, D)(flat_idx, table)
    return out.reshape(B, H, D)
